# speedup vs baseline: 3.6436x; 3.6436x over previous
import functools
import jax
import jax.numpy as jnp
from jax.experimental import pallas as pl

N = 100000
E = 1600000
H, C = 8, 16
IN = 16
HC = H * C
BLK = 2000


def _dense_front_body(x_ref, w_ref, asm_ref, adm_ref, h_ref, as_ref, ad_ref, wi_ref):
    x = x_ref[...]
    h = jnp.dot(x, w_ref[...], preferred_element_type=jnp.float32)
    a_src = jnp.dot(h, asm_ref[...], preferred_element_type=jnp.float32)
    a_dst = jnp.dot(h, adm_ref[...], preferred_element_type=jnp.float32)
    e = a_src + a_dst
    e = jnp.where(e >= 0, e, 0.2 * e)
    wi_ref[...] = jnp.exp(e)
    h_ref[...] = h
    as_ref[...] = a_src
    ad_ref[...] = a_dst


def _dense_front(x, W, As, Ad):
    grid = (N // BLK,)
    return pl.pallas_call(
        _dense_front_body,
        grid=grid,
        in_specs=[
            pl.BlockSpec((BLK, IN), lambda i: (i, 0)),
            pl.BlockSpec((IN, HC), lambda i: (0, 0)),
            pl.BlockSpec((HC, H), lambda i: (0, 0)),
            pl.BlockSpec((HC, H), lambda i: (0, 0)),
        ],
        out_specs=[
            pl.BlockSpec((BLK, HC), lambda i: (i, 0)),
            pl.BlockSpec((BLK, H), lambda i: (i, 0)),
            pl.BlockSpec((BLK, H), lambda i: (i, 0)),
            pl.BlockSpec((BLK, H), lambda i: (i, 0)),
        ],
        out_shape=[
            jax.ShapeDtypeStruct((N, HC), jnp.float32),
            jax.ShapeDtypeStruct((N, H), jnp.float32),
            jax.ShapeDtypeStruct((N, H), jnp.float32),
            jax.ShapeDtypeStruct((N, H), jnp.float32),
        ],
    )(x, W, As, Ad)


def _dense_back_body(num_ref, den_ref, wih_ref, bias_ref, fcw_ref, fcb_ref, o_ref):
    num = num_ref[...] + wih_ref[...]
    den = den_ref[...]
    den_e = jnp.repeat(den, C, axis=1)
    val = num / den_e + bias_ref[...]
    val = jnp.maximum(val, 0.0)
    logits = jnp.dot(val, fcw_ref[...], preferred_element_type=jnp.float32) + fcb_ref[...]
    m = jnp.max(logits, axis=1, keepdims=True)
    z = logits - m
    lse = jnp.log(jnp.sum(jnp.exp(z), axis=1, keepdims=True))
    o_ref[...] = z - lse


def _dense_back(num, den, wih, bias, fc_W, fc_b):
    grid = (N // BLK,)
    return pl.pallas_call(
        _dense_back_body,
        grid=grid,
        in_specs=[
            pl.BlockSpec((BLK, HC), lambda i: (i, 0)),
            pl.BlockSpec((BLK, H), lambda i: (i, 0)),
            pl.BlockSpec((BLK, HC), lambda i: (i, 0)),
            pl.BlockSpec((1, HC), lambda i: (0, 0)),
            pl.BlockSpec((HC, 5), lambda i: (0, 0)),
            pl.BlockSpec((1, 5), lambda i: (0, 0)),
        ],
        out_specs=pl.BlockSpec((BLK, 5), lambda i: (i, 0)),
        out_shape=jax.ShapeDtypeStruct((N, 5), jnp.float32),
    )(num, den, wih, bias, fc_W, fc_b)


def kernel(x, edge_index, W, att_src, att_dst, bias, fc_W, fc_b):
    src = edge_index[0]
    dst = edge_index[1]
    eyeH = jnp.eye(H, dtype=jnp.float32)
    As = (att_src.reshape(H, C, 1) * eyeH[:, None, :]).reshape(HC, H)
    Ad = (att_dst.reshape(H, C, 1) * eyeH[:, None, :]).reshape(HC, H)

    h, a_src, a_dst, w_loop = _dense_front(x, W, As, Ad)

    e = a_src[src] + a_dst[dst]
    e = jnp.where(e >= 0, e, 0.2 * e)
    w = jnp.exp(e)
    den = jax.ops.segment_sum(w, dst, num_segments=N) + w_loop
    h3 = h.reshape(N, H, C)
    msgs = (h3[src] * w[:, :, None]).reshape(E, HC)
    num = jax.ops.segment_sum(msgs, dst, num_segments=N)
    wih = (h3 * w_loop[:, :, None]).reshape(N, HC)

    return _dense_back(num, den, wih, bias.reshape(1, HC), fc_W, fc_b.reshape(1, 5))


# speedup vs baseline: 3.9613x; 1.0872x over previous
import functools
import jax
import jax.numpy as jnp
from jax import lax
from jax.experimental import pallas as pl
from jax.experimental.pallas import tpu as pltpu
from jax.experimental.pallas import tpu_sc as plsc

N = 100000
E = 1600000
H, C = 8, 16
IN = 16
HC = H * C
BLK = 2000

NW = 32
EB = 1024
NB1 = 49
EP = NW * NB1 * EB


def _dense_front_body(x_ref, w_ref, asm_ref, adm_ref, h_ref, t1_ref, t2_ref, wi_ref):
    x = x_ref[...]
    h = jnp.dot(x, w_ref[...], preferred_element_type=jnp.float32)
    a_src = jnp.dot(h, asm_ref[...], preferred_element_type=jnp.float32)
    a_dst = jnp.dot(h, adm_ref[...], preferred_element_type=jnp.float32)
    e = a_src + a_dst
    e = jnp.where(e >= 0, e, 0.2 * e)
    wi_ref[...] = jnp.exp(e)
    h_ref[...] = h
    t1_ref[...] = jnp.concatenate([a_src, a_dst], axis=1)
    t2_ref[...] = jnp.concatenate([a_dst, a_src], axis=1)


def _dense_front(x, W, As, Ad):
    grid = (N // BLK,)
    return pl.pallas_call(
        _dense_front_body,
        grid=grid,
        in_specs=[
            pl.BlockSpec((BLK, IN), lambda i: (i, 0)),
            pl.BlockSpec((IN, HC), lambda i: (0, 0)),
            pl.BlockSpec((HC, H), lambda i: (0, 0)),
            pl.BlockSpec((HC, H), lambda i: (0, 0)),
        ],
        out_specs=[
            pl.BlockSpec((BLK, HC), lambda i: (i, 0)),
            pl.BlockSpec((BLK, 2 * H), lambda i: (i, 0)),
            pl.BlockSpec((BLK, 2 * H), lambda i: (i, 0)),
            pl.BlockSpec((BLK, H), lambda i: (i, 0)),
        ],
        out_shape=[
            jax.ShapeDtypeStruct((N, HC), jnp.float32),
            jax.ShapeDtypeStruct((N, 2 * H), jnp.float32),
            jax.ShapeDtypeStruct((N, 2 * H), jnp.float32),
            jax.ShapeDtypeStruct((N, H), jnp.float32),
        ],
    )(x, W, As, Ad)


def _sc_phase1(src_pad, dst_pad, T1, T2):
    src2d = src_pad.reshape(EP // 128, 128)
    dst2d = dst_pad.reshape(EP // 128, 128)
    mesh = plsc.VectorSubcoreMesh(core_axis_name="c", subcore_axis_name="s",
                                  num_cores=2, num_subcores=16)

    @functools.partial(
        pl.kernel,
        out_type=jax.ShapeDtypeStruct((EP, 16), jnp.float32),
        mesh=mesh,
        scratch_types=[
            pltpu.VMEM((8, 128), jnp.int32),
            pltpu.VMEM((8, 128), jnp.int32),
            pltpu.VMEM((EB, 16), jnp.float32),
            pltpu.SemaphoreType.DMA,
        ],
        compiler_params=pltpu.CompilerParams(use_tc_tiling_on_sc=False),
    )
    def k(src_hbm, dst_hbm, t1_hbm, t2_hbm, w_hbm, sidx, didx, buf, sem):
        wid = lax.axis_index("s") * 2 + lax.axis_index("c")

        def blk(b, carry):
            g = wid * NB1 + b
            pltpu.sync_copy(src_hbm.at[pl.ds(g * 8, 8)], sidx)
            pltpu.sync_copy(dst_hbm.at[pl.ds(g * 8, 8)], didx)
            descs = [
                pltpu.async_copy(t1_hbm.at[sidx.at[j]],
                                 buf.at[pl.ds(j * 128, 128)], sem)
                for j in range(8)
            ]
            for d in descs:
                d.wait()
            descs = [
                pltpu.async_copy(t2_hbm.at[didx.at[j]],
                                 buf.at[pl.ds(j * 128, 128)], sem, add=True)
                for j in range(8)
            ]
            for d in descs:
                d.wait()

            def cbody(iv, c2):
                for u in range(16):
                    i = iv * 16 + u
                    v = buf[i, :]
                    v = jnp.where(v >= 0, v, 0.2 * v)
                    buf[i, :] = jnp.exp(v)
                return c2

            lax.fori_loop(0, EB // 16, cbody, 0)
            pltpu.sync_copy(buf, w_hbm.at[pl.ds(g * EB, EB)])
            return carry

        lax.fori_loop(0, NB1, blk, 0)

    return k(src2d, dst2d, T1, T2)


def _dense_back_body(num_ref, den_ref, wih_ref, bias_ref, fcw_ref, fcb_ref, o_ref):
    num = num_ref[...] + wih_ref[...]
    den = den_ref[...]
    den_e = jnp.repeat(den, C, axis=1)
    val = num / den_e + bias_ref[...]
    val = jnp.maximum(val, 0.0)
    logits = jnp.dot(val, fcw_ref[...], preferred_element_type=jnp.float32) + fcb_ref[...]
    m = jnp.max(logits, axis=1, keepdims=True)
    z = logits - m
    lse = jnp.log(jnp.sum(jnp.exp(z), axis=1, keepdims=True))
    o_ref[...] = z - lse


def _dense_back(num, den, wih, bias, fc_W, fc_b):
    grid = (N // BLK,)
    return pl.pallas_call(
        _dense_back_body,
        grid=grid,
        in_specs=[
            pl.BlockSpec((BLK, HC), lambda i: (i, 0)),
            pl.BlockSpec((BLK, H), lambda i: (i, 0)),
            pl.BlockSpec((BLK, HC), lambda i: (i, 0)),
            pl.BlockSpec((1, HC), lambda i: (0, 0)),
            pl.BlockSpec((HC, 5), lambda i: (0, 0)),
            pl.BlockSpec((1, 5), lambda i: (0, 0)),
        ],
        out_specs=pl.BlockSpec((BLK, 5), lambda i: (i, 0)),
        out_shape=jax.ShapeDtypeStruct((N, 5), jnp.float32),
    )(num, den, wih, bias, fc_W, fc_b)


def kernel(x, edge_index, W, att_src, att_dst, bias, fc_W, fc_b):
    src = edge_index[0]
    dst = edge_index[1]
    eyeH = jnp.eye(H, dtype=jnp.float32)
    As = (att_src.reshape(H, C, 1) * eyeH[:, None, :]).reshape(HC, H)
    Ad = (att_dst.reshape(H, C, 1) * eyeH[:, None, :]).reshape(HC, H)

    h, T1, T2, w_loop = _dense_front(x, W, As, Ad)

    pad = EP - E
    padi = (jnp.arange(pad, dtype=jnp.int32) * 8191) % N
    src_pad = jnp.concatenate([src, padi])
    dst_pad = jnp.concatenate([dst, padi])
    w_full = _sc_phase1(src_pad, dst_pad, T1, T2)
    w = w_full[:E, :H]

    den = jax.ops.segment_sum(w, dst, num_segments=N) + w_loop
    h3 = h.reshape(N, H, C)
    msgs = (h3[src] * w[:, :, None]).reshape(E, HC)
    num = jax.ops.segment_sum(msgs, dst, num_segments=N)
    wih = (h3 * w_loop[:, :, None]).reshape(N, HC)

    return _dense_back(num, den, wih, bias.reshape(1, HC), fc_W, fc_b.reshape(1, 5))


# speedup vs baseline: 65.8172x; 16.6150x over previous
import functools
import jax
import jax.numpy as jnp
from jax import lax
from jax.experimental import pallas as pl
from jax.experimental.pallas import tpu as pltpu
from jax.experimental.pallas import tpu_sc as plsc

N = 100000
E = 1600000
H, C = 8, 16
IN = 16
HC = H * C
BLK = 2000

NW = 32
EB = 1024
NB1 = 49
EP = NW * NB1 * EB
EPW = EP // NW
ERW = E // NW

CH = 4096
NR = -(-N // CH)
NP = NR * CH
RPS = CH // 16
B2 = 128
ECAP = 4096


def _dense_front_body(x_ref, w_ref, asm_ref, adm_ref, h_ref, t1_ref, t2_ref, wi_ref):
    x = x_ref[...]
    h = jnp.dot(x, w_ref[...], preferred_element_type=jnp.float32)
    a_src = jnp.dot(h, asm_ref[...], preferred_element_type=jnp.float32)
    a_dst = jnp.dot(h, adm_ref[...], preferred_element_type=jnp.float32)
    e = a_src + a_dst
    e = jnp.where(e >= 0, e, 0.2 * e)
    wi_ref[...] = jnp.exp(e)
    h_ref[...] = h
    t1_ref[...] = jnp.concatenate([a_src, a_dst], axis=1)
    t2_ref[...] = jnp.concatenate([a_dst, a_src], axis=1)


def _dense_front(x, W, As, Ad):
    grid = (N // BLK,)
    return pl.pallas_call(
        _dense_front_body,
        grid=grid,
        in_specs=[
            pl.BlockSpec((BLK, IN), lambda i: (i, 0)),
            pl.BlockSpec((IN, HC), lambda i: (0, 0)),
            pl.BlockSpec((HC, H), lambda i: (0, 0)),
            pl.BlockSpec((HC, H), lambda i: (0, 0)),
        ],
        out_specs=[
            pl.BlockSpec((BLK, HC), lambda i: (i, 0)),
            pl.BlockSpec((BLK, 2 * H), lambda i: (i, 0)),
            pl.BlockSpec((BLK, 2 * H), lambda i: (i, 0)),
            pl.BlockSpec((BLK, H), lambda i: (i, 0)),
        ],
        out_shape=[
            jax.ShapeDtypeStruct((N, HC), jnp.float32),
            jax.ShapeDtypeStruct((N, 2 * H), jnp.float32),
            jax.ShapeDtypeStruct((N, 2 * H), jnp.float32),
            jax.ShapeDtypeStruct((N, H), jnp.float32),
        ],
    )(x, W, As, Ad)


def _sc_phase1(src_pad, dst_pad, T1, T2):
    src2d = src_pad.reshape(EP // 128, 128)
    dst2d = dst_pad.reshape(EP // 128, 128)
    mesh = plsc.VectorSubcoreMesh(core_axis_name="c", subcore_axis_name="s",
                                  num_cores=2, num_subcores=16)

    @functools.partial(
        pl.kernel,
        out_type=jax.ShapeDtypeStruct((EP, 16), jnp.float32),
        mesh=mesh,
        scratch_types=[
            pltpu.VMEM((8, 128), jnp.int32),
            pltpu.VMEM((8, 128), jnp.int32),
            pltpu.VMEM((EB, 16), jnp.float32),
            pltpu.SemaphoreType.DMA,
        ],
        compiler_params=pltpu.CompilerParams(use_tc_tiling_on_sc=False),
    )
    def k(src_hbm, dst_hbm, t1_hbm, t2_hbm, w_hbm, sidx, didx, buf, sem):
        wid = lax.axis_index("s") * 2 + lax.axis_index("c")

        def blk(b, carry):
            g = wid * NB1 + b
            pltpu.sync_copy(src_hbm.at[pl.ds(g * 8, 8)], sidx)
            pltpu.sync_copy(dst_hbm.at[pl.ds(g * 8, 8)], didx)
            descs = [
                pltpu.async_copy(t1_hbm.at[sidx.at[j]],
                                 buf.at[pl.ds(j * 128, 128)], sem)
                for j in range(8)
            ]
            for d in descs:
                d.wait()
            descs = [
                pltpu.async_copy(t2_hbm.at[didx.at[j]],
                                 buf.at[pl.ds(j * 128, 128)], sem, add=True)
                for j in range(8)
            ]
            for d in descs:
                d.wait()

            def cbody(iv, c2):
                for u in range(16):
                    i = iv * 16 + u
                    v = buf[i, :]
                    v = jnp.where(v >= 0, v, 0.2 * v)
                    buf[i, :] = jnp.exp(v)
                return c2

            lax.fori_loop(0, EB // 16, cbody, 0)

            @pl.when(b == NB1 - 1)
            def _():
                def zpad(i, c2):
                    buf[(ERW - (NB1 - 1) * EB) + i, :] = jnp.zeros((16,), jnp.float32)
                    return c2
                lax.fori_loop(0, EB - (ERW - (NB1 - 1) * EB), zpad, 0)

            pltpu.sync_copy(buf, w_hbm.at[pl.ds(g * EB, EB)])
            return carry

        lax.fori_loop(0, NB1, blk, 0)

    return k(src2d, dst2d, T1, T2)


def _sc_phase2(src_pad, dst_pad, w_full, h):
    dst2d = dst_pad.reshape(EP // 16, 16)
    mesh = plsc.VectorSubcoreMesh(core_axis_name="c", subcore_axis_name="s",
                                  num_cores=2, num_subcores=16)

    @functools.partial(
        pl.kernel,
        out_type=[
            jax.ShapeDtypeStruct((2, NP, HC), jnp.float32),
            jax.ShapeDtypeStruct((2, NP, 16), jnp.float32),
        ],
        mesh=mesh,
        scratch_types=[
            pltpu.VMEM((EP // NW // 16, 16), jnp.int32),
            pltpu.VMEM((ECAP,), jnp.int32),
            pltpu.VMEM((B2, 16), jnp.float32),
            pltpu.VMEM((B2,), jnp.int32),
            pltpu.VMEM((B2, HC), jnp.float32),
            pltpu.VMEM((B2,), jnp.int32),
            pltpu.VMEM((64, HC), jnp.float32),
            pltpu.VMEM((128, 16), jnp.float32),
            pltpu.VMEM_SHARED((CH, HC), jnp.float32),
            pltpu.VMEM_SHARED((CH, 16), jnp.float32),
            pltpu.SemaphoreType.DMA,
        ],
        compiler_params=pltpu.CompilerParams(use_tc_tiling_on_sc=False,
                                             needs_layout_passes=False),
    )
    def k(src_hbm, dst_hbm, w_hbm, h_hbm, outp_hbm, denp_hbm,
          dstr, ebuf, wbuf, sbuf, hbuf, dlbuf, zbuf, zdbuf, out_sh, den_sh, sem):
        cid = lax.axis_index("c")
        sid = lax.axis_index("s")
        wid = sid * 2 + cid
        ebase = wid * EPW
        iota = lax.iota(jnp.int32, 16)

        def zv(buf, nrow, ncol):
            def zi(i, c):
                for kk in range(ncol // 16):
                    buf[i, pl.ds(kk * 16, 16)] = jnp.zeros((16,), jnp.float32)
                return c
            lax.fori_loop(0, nrow, zi, 0)

        zv(zbuf, 64, HC)
        zv(zdbuf, 128, 16)
        pltpu.sync_copy(dst_hbm.at[pl.ds(wid * (EPW // 16), EPW // 16)], dstr)

        def range_body(r, carry):
            lo = r * CH
            row0 = sid * RPS
            for kk in range(RPS // 64):
                pltpu.sync_copy(zbuf, out_sh.at[pl.ds(row0 + kk * 64, 64)])
            if RPS % 64:
                pltpu.sync_copy(zbuf.at[pl.ds(0, 16)],
                                out_sh.at[pl.ds(row0 + RPS - 16, 16)])
            for kk in range(RPS // 128):
                pltpu.sync_copy(zdbuf, den_sh.at[pl.ds(row0 + kk * 128, 128)])
            if RPS % 128:
                pltpu.sync_copy(zdbuf, den_sh.at[pl.ds(row0 + RPS - 128, 128)])
            plsc.subcore_barrier()

            lov = jnp.full((16,), lo, jnp.int32)
            ebasev = jnp.full((16,), ebase, jnp.int32)
            erealv = ebasev + ERW

            def fbody(v, cnt):
                d = dstr[v, :]
                eidv = iota + jnp.full((16,), v * 16 + ebase, jnp.int32)
                m = (d >= lov) & (d < lov + CH) & (eidv < erealv)
                cs = plsc.cumsum(m.astype(jnp.int32))
                pos = jnp.full((16,), cnt, jnp.int32) + cs - 1
                plsc.store_scatter(ebuf, [pos], eidv, mask=m)
                return cnt + cs[15]
            cnt = lax.fori_loop(0, EPW // 16, fbody, jnp.int32(0))
            padv = erealv + (iota & 7) * 8
            for kk in range(B2 // 16):
                ebuf[pl.ds(cnt + kk * 16, 16)] = padv
            nblk = (cnt + (B2 - 1)) // B2

            def bbody(b, c2):
                idxs = ebuf.at[pl.ds(b * B2, B2)]
                d1 = pltpu.async_copy(w_hbm.at[idxs], wbuf, sem)
                d2 = pltpu.async_copy(src_hbm.at[idxs], sbuf, sem)
                d1.wait()
                d2.wait()
                d3 = pltpu.async_copy(h_hbm.at[sbuf], hbuf, sem)
                cntv = jnp.full((16,), cnt, jnp.int32)
                for j in range(B2 // 16):
                    eidv = ebuf[pl.ds(b * B2 + j * 16, 16)]
                    el = eidv - ebasev
                    dg = plsc.load_gather(dstr, [el >> 4, el & 15])
                    gpos = iota + jnp.full((16,), b * B2 + j * 16, jnp.int32)
                    dl = jnp.where(gpos < cntv, dg - lov, iota & 7)
                    dlbuf[pl.ds(j * 16, 16)] = dl
                d3.wait()

                def mbody(i, c3):
                    wrow = wbuf[i, :]
                    for hh in range(H):
                        hidx = jnp.full((16,), hh, jnp.int32)
                        wb = wrow.at[hidx].get(mode="promise_in_bounds")
                        hv = hbuf[i, pl.ds(hh * C, C)]
                        hbuf[i, pl.ds(hh * C, C)] = hv * wb
                    return c3
                lax.fori_loop(0, B2, mbody, 0)
                pltpu.sync_copy(wbuf, den_sh.at[dlbuf], add=True)
                pltpu.sync_copy(hbuf, out_sh.at[dlbuf], add=True)
                return c2
            lax.fori_loop(0, nblk, bbody, 0)
            plsc.subcore_barrier()

            vrow = CH // 16
            pltpu.sync_copy(out_sh.at[pl.ds(sid * vrow, vrow)],
                            outp_hbm.at[cid, pl.ds(lo + sid * vrow, vrow)])
            pltpu.sync_copy(den_sh.at[pl.ds(sid * vrow, vrow)],
                            denp_hbm.at[cid, pl.ds(lo + sid * vrow, vrow)])
            plsc.subcore_barrier()
            return carry

        lax.fori_loop(0, NR, range_body, 0)

    return k(src_pad, dst2d, w_full, h)


def _dense_back_body(o0_ref, o1_ref, d0_ref, d1_ref, wl_ref, h_ref,
                     bias_ref, fcw_ref, fcb_ref, o_ref):
    wl = wl_ref[...]
    den = d0_ref[0][:, :H] + d1_ref[0][:, :H] + wl
    num = o0_ref[0] + o1_ref[0] + h_ref[...] * jnp.repeat(wl, C, axis=1)
    val = num / jnp.repeat(den, C, axis=1) + bias_ref[...]
    val = jnp.maximum(val, 0.0)
    logits = jnp.dot(val, fcw_ref[...], preferred_element_type=jnp.float32) + fcb_ref[...]
    m = jnp.max(logits, axis=1, keepdims=True)
    z = logits - m
    lse = jnp.log(jnp.sum(jnp.exp(z), axis=1, keepdims=True))
    o_ref[...] = z - lse


def _dense_back(outp, denp, w_loop, h, bias, fc_W, fc_b):
    grid = (N // BLK,)
    return pl.pallas_call(
        _dense_back_body,
        grid=grid,
        in_specs=[
            pl.BlockSpec((1, BLK, HC), lambda i: (0, i, 0)),
            pl.BlockSpec((1, BLK, HC), lambda i: (1, i, 0)),
            pl.BlockSpec((1, BLK, 16), lambda i: (0, i, 0)),
            pl.BlockSpec((1, BLK, 16), lambda i: (1, i, 0)),
            pl.BlockSpec((BLK, H), lambda i: (i, 0)),
            pl.BlockSpec((BLK, HC), lambda i: (i, 0)),
            pl.BlockSpec((1, HC), lambda i: (0, 0)),
            pl.BlockSpec((HC, 5), lambda i: (0, 0)),
            pl.BlockSpec((1, 5), lambda i: (0, 0)),
        ],
        out_specs=pl.BlockSpec((BLK, 5), lambda i: (i, 0)),
        out_shape=jax.ShapeDtypeStruct((N, 5), jnp.float32),
    )(outp, outp, denp, denp, w_loop, h, bias, fc_W, fc_b)


def kernel(x, edge_index, W, att_src, att_dst, bias, fc_W, fc_b):
    src = edge_index[0]
    dst = edge_index[1]
    eyeH = jnp.eye(H, dtype=jnp.float32)
    As = (att_src.reshape(H, C, 1) * eyeH[:, None, :]).reshape(HC, H)
    Ad = (att_dst.reshape(H, C, 1) * eyeH[:, None, :]).reshape(HC, H)

    h, T1, T2, w_loop = _dense_front(x, W, As, Ad)

    padi = ((jnp.arange(NW * (EPW - ERW), dtype=jnp.int32) * 8191) % N
            ).reshape(NW, EPW - ERW)
    src_pad = jnp.concatenate([src.reshape(NW, ERW), padi], axis=1).reshape(EP)
    dst_pad = jnp.concatenate([dst.reshape(NW, ERW), padi], axis=1).reshape(EP)
    w_full = _sc_phase1(src_pad, dst_pad, T1, T2)
    outp, denp = _sc_phase2(src_pad, dst_pad, w_full, h)

    return _dense_back(outp, denp, w_loop, h,
                       bias.reshape(1, HC), fc_W, fc_b.reshape(1, 5))


# speedup vs baseline: 78.4870x; 1.1925x over previous
import functools
import jax
import jax.numpy as jnp
from jax import lax
from jax.experimental import pallas as pl
from jax.experimental.pallas import tpu as pltpu
from jax.experimental.pallas import tpu_sc as plsc

N = 100000
E = 1600000
H, C = 8, 16
IN = 16
HC = H * C
BLK = 2000

NW = 32
EB = 1024
NB1 = 49
EP = NW * NB1 * EB
EPW = EP // NW
ERW = E // NW

CH = 5120
NR = -(-N // CH)
NP = NR * CH
RPS = CH // 16
B2 = 128
ECAP = 3840
NQ = ECAP // B2
FCH = 112
NCHK = EPW // (FCH * 16)


def _dense_front_body(x_ref, w_ref, asm_ref, adm_ref, h_ref, t1_ref, t2_ref, wi_ref):
    x = x_ref[...]
    h = jnp.dot(x, w_ref[...], preferred_element_type=jnp.float32)
    a_src = jnp.dot(h, asm_ref[...], preferred_element_type=jnp.float32)
    a_dst = jnp.dot(h, adm_ref[...], preferred_element_type=jnp.float32)
    e = a_src + a_dst
    e = jnp.where(e >= 0, e, 0.2 * e)
    wi_ref[...] = jnp.exp(e)
    h_ref[...] = h
    t1_ref[...] = jnp.concatenate([a_src, a_dst], axis=1)
    t2_ref[...] = jnp.concatenate([a_dst, a_src], axis=1)


def _dense_front(x, W, As, Ad):
    grid = (N // BLK,)
    return pl.pallas_call(
        _dense_front_body,
        grid=grid,
        in_specs=[
            pl.BlockSpec((BLK, IN), lambda i: (i, 0)),
            pl.BlockSpec((IN, HC), lambda i: (0, 0)),
            pl.BlockSpec((HC, H), lambda i: (0, 0)),
            pl.BlockSpec((HC, H), lambda i: (0, 0)),
        ],
        out_specs=[
            pl.BlockSpec((BLK, HC), lambda i: (i, 0)),
            pl.BlockSpec((BLK, 2 * H), lambda i: (i, 0)),
            pl.BlockSpec((BLK, 2 * H), lambda i: (i, 0)),
            pl.BlockSpec((BLK, H), lambda i: (i, 0)),
        ],
        out_shape=[
            jax.ShapeDtypeStruct((N, HC), jnp.float32),
            jax.ShapeDtypeStruct((N, 2 * H), jnp.float32),
            jax.ShapeDtypeStruct((N, 2 * H), jnp.float32),
            jax.ShapeDtypeStruct((N, H), jnp.float32),
        ],
    )(x, W, As, Ad)


def _sc_phase1(src_pad, dst_pad, T1, T2):
    src2d = src_pad.reshape(EP // 128, 128)
    dst2d = dst_pad.reshape(EP // 128, 128)
    mesh = plsc.VectorSubcoreMesh(core_axis_name="c", subcore_axis_name="s",
                                  num_cores=2, num_subcores=16)

    @functools.partial(
        pl.kernel,
        out_type=jax.ShapeDtypeStruct((EP, 16), jnp.float32),
        mesh=mesh,
        scratch_types=[
            pltpu.VMEM((8, 128), jnp.int32),
            pltpu.VMEM((8, 128), jnp.int32),
            pltpu.VMEM((EB, 16), jnp.float32),
            pltpu.SemaphoreType.DMA,
        ],
        compiler_params=pltpu.CompilerParams(use_tc_tiling_on_sc=False),
    )
    def k(src_hbm, dst_hbm, t1_hbm, t2_hbm, w_hbm, sidx, didx, buf, sem):
        wid = lax.axis_index("s") * 2 + lax.axis_index("c")

        def blk(b, carry):
            g = wid * NB1 + b
            pltpu.sync_copy(src_hbm.at[pl.ds(g * 8, 8)], sidx)
            pltpu.sync_copy(dst_hbm.at[pl.ds(g * 8, 8)], didx)
            descs = [
                pltpu.async_copy(t1_hbm.at[sidx.at[j]],
                                 buf.at[pl.ds(j * 128, 128)], sem)
                for j in range(8)
            ]
            for d in descs:
                d.wait()
            descs = [
                pltpu.async_copy(t2_hbm.at[didx.at[j]],
                                 buf.at[pl.ds(j * 128, 128)], sem, add=True)
                for j in range(8)
            ]
            for d in descs:
                d.wait()

            def cbody(iv, c2):
                for u in range(16):
                    i = iv * 16 + u
                    v = buf[i, :]
                    v = jnp.where(v >= 0, v, 0.2 * v)
                    buf[i, :] = jnp.exp(v)
                return c2

            lax.fori_loop(0, EB // 16, cbody, 0)

            @pl.when(b == NB1 - 1)
            def _():
                def zpad(i, c2):
                    buf[(ERW - (NB1 - 1) * EB) + i, :] = jnp.zeros((16,), jnp.float32)
                    return c2
                lax.fori_loop(0, EB - (ERW - (NB1 - 1) * EB), zpad, 0)

            pltpu.sync_copy(buf, w_hbm.at[pl.ds(g * EB, EB)])
            return carry

        lax.fori_loop(0, NB1, blk, 0)

    return k(src2d, dst2d, T1, T2)


def _sc_phase2(src_pad, dst_pad, w_full, h):
    dst2d = dst_pad.reshape(EP // 16, 16)
    mesh = plsc.VectorSubcoreMesh(core_axis_name="c", subcore_axis_name="s",
                                  num_cores=2, num_subcores=16)

    @functools.partial(
        pl.kernel,
        out_type=[
            jax.ShapeDtypeStruct((2, NP, HC), jnp.float32),
            jax.ShapeDtypeStruct((2, NP, 16), jnp.float32),
        ],
        mesh=mesh,
        scratch_types=[
            pltpu.VMEM((ECAP,), jnp.int32),
            pltpu.VMEM((NQ, B2), jnp.int32),
            pltpu.VMEM((NQ, B2), jnp.int32),
            pltpu.VMEM((NQ, B2), jnp.int32),
            pltpu.VMEM((FCH * 16,), jnp.int32),
            pltpu.VMEM((FCH * 16,), jnp.int32),
            pltpu.VMEM((B2, 16), jnp.float32),
            pltpu.VMEM((B2, 16), jnp.float32),
            pltpu.VMEM((B2, HC), jnp.float32),
            pltpu.VMEM((B2, HC), jnp.float32),
            pltpu.VMEM((16, HC), jnp.float32),
            pltpu.VMEM((64, 16), jnp.float32),
            pltpu.VMEM_SHARED((CH, HC), jnp.float32),
            pltpu.VMEM_SHARED((CH, 16), jnp.float32),
            pltpu.SemaphoreType.DMA,
            pltpu.SemaphoreType.DMA,
            pltpu.SemaphoreType.DMA,
            pltpu.SemaphoreType.DMA,
            pltpu.SemaphoreType.DMA,
            pltpu.SemaphoreType.DMA,
            pltpu.SemaphoreType.DMA,
        ],
        compiler_params=pltpu.CompilerParams(use_tc_tiling_on_sc=False,
                                             needs_layout_passes=False),
    )
    def k(src_hbm, dst_flat_hbm, w_hbm, h_hbm, outp_hbm, denp_hbm,
          ebuf, srng, dsr, dlr, fb0, fb1, wb0, wb1, hb0, hb1,
          zbuf, zdbuf, out_sh, den_sh,
          psem, fs0, fs1, gs0, gs1, ss0, ss1):
        fbufs = (fb0, fb1)
        fsems = (fs0, fs1)
        wbufs = (wb0, wb1)
        hbufs = (hb0, hb1)
        gsems = (gs0, gs1)
        ssems = (ss0, ss1)
        cid = lax.axis_index("c")
        sid = lax.axis_index("s")
        wid = sid * 2 + cid
        ebase = wid * EPW
        iota = lax.iota(jnp.int32, 16)

        def zv(buf, nrow, ncol):
            def zi(i, c):
                for kk in range(ncol // 16):
                    buf[i, pl.ds(kk * 16, 16)] = jnp.zeros((16,), jnp.float32)
                return c
            lax.fori_loop(0, nrow, zi, 0)

        zv(zbuf, 16, HC)
        zv(zdbuf, 64, 16)
        fchunk = lambda c: dst_flat_hbm.at[pl.ds(wid * EPW + c * FCH * 16,
                                                 FCH * 16)]

        def range_body(r, carry):
            lo = r * CH
            row0 = sid * RPS
            for kk in range(RPS // 16):
                pltpu.sync_copy(zbuf, out_sh.at[pl.ds(row0 + kk * 16, 16)])
            for kk in range(RPS // 64):
                pltpu.sync_copy(zdbuf, den_sh.at[pl.ds(row0 + kk * 64, 64)])
            plsc.subcore_barrier()

            lov = jnp.full((16,), lo, jnp.int32)
            ebasev = jnp.full((16,), ebase, jnp.int32)
            erealv = ebasev + ERW

            for u in range(2):
                pltpu.async_copy(fchunk(u), fbufs[u], fsems[u])

            def fchunk_body(c, u, cnt):
                pltpu.make_async_copy(fchunk(c), fbufs[u], fsems[u]).wait()

                def fbody(v, cnt2):
                    d = fbufs[u][pl.ds(v * 16, 16)]
                    eidv = iota + jnp.full(
                        (16,), c * FCH * 16 + v * 16 + ebase, jnp.int32)
                    m = (d >= lov) & (d < lov + CH) & (eidv < erealv)
                    cs = plsc.cumsum(m.astype(jnp.int32))
                    pos = jnp.full((16,), cnt2, jnp.int32) + cs - 1
                    plsc.store_scatter(ebuf, [pos], eidv, mask=m)
                    return cnt2 + cs[15]
                cnt = lax.fori_loop(0, FCH, fbody, cnt)

                @pl.when(c + 2 < NCHK)
                def _():
                    pltpu.async_copy(fchunk(c + 2), fbufs[u], fsems[u])
                return cnt

            def fpair(p, cnt):
                for u in range(2):
                    cnt = fchunk_body(p * 2 + u, u, cnt)
                return cnt
            cnt = lax.fori_loop(0, NCHK // 2, fpair, jnp.int32(0))
            padv = erealv + (iota & 7) * 8
            for kk in range(B2 // 16):
                ebuf[pl.ds(cnt + kk * 16, 16)] = padv
            nblk = (cnt + (B2 - 1)) // B2
            cntv = jnp.full((16,), cnt, jnp.int32)

            for q in range(NQ):
                @pl.when(q < nblk)
                def _():
                    pltpu.async_copy(src_hbm.at[ebuf.at[pl.ds(q * B2, B2)]],
                                     srng.at[q], psem)
                    pltpu.async_copy(dst_flat_hbm.at[ebuf.at[pl.ds(q * B2, B2)]],
                                     dsr.at[q], psem)
            for q in range(NQ):
                @pl.when(q < nblk)
                def _():
                    pltpu.make_async_copy(src_hbm.at[ebuf.at[pl.ds(q * B2, B2)]],
                                          srng.at[q], psem).wait()
                    pltpu.make_async_copy(dst_flat_hbm.at[ebuf.at[pl.ds(q * B2, B2)]],
                                          dsr.at[q], psem).wait()

            def dbody(bq, c2):
                for j in range(B2 // 16):
                    dg = dsr[bq, pl.ds(j * 16, 16)]
                    gpos = iota + jnp.full((16,), bq * B2 + j * 16, jnp.int32)
                    dl = jnp.where(gpos < cntv, dg - lov, iota & 7)
                    dlr[bq, pl.ds(j * 16, 16)] = dl
                return c2
            lax.fori_loop(0, nblk, dbody, 0)

            def issue_g(u, b):
                pltpu.async_copy(w_hbm.at[ebuf.at[pl.ds(b * B2, B2)]],
                                 wbufs[u], gsems[u])
                pltpu.async_copy(h_hbm.at[srng.at[b]], hbufs[u], gsems[u])

            def wait_g(u, b):
                pltpu.make_async_copy(w_hbm.at[ebuf.at[pl.ds(b * B2, B2)]],
                                      wbufs[u], gsems[u]).wait()
                pltpu.make_async_copy(h_hbm.at[srng.at[b]], hbufs[u],
                                      gsems[u]).wait()

            def mul(u):
                wbu, hbu = wbufs[u], hbufs[u]

                def mbody(i2, c3):
                    for e in range(2):
                        i = i2 * 2 + e
                        wrow = wbu[i, :]
                        for hh in range(H):
                            hidx = jnp.full((16,), hh, jnp.int32)
                            wb = wrow.at[hidx].get(mode="promise_in_bounds")
                            hv = hbu[i, pl.ds(hh * C, C)]
                            hbu[i, pl.ds(hh * C, C)] = hv * wb
                    return c3
                lax.fori_loop(0, B2 // 2, mbody, 0)

            def issue_s(u, b):
                pltpu.async_copy(wbufs[u], den_sh.at[dlr.at[b]], ssems[u],
                                 add=True)
                pltpu.async_copy(hbufs[u], out_sh.at[dlr.at[b]], ssems[u],
                                 add=True)

            def wait_s(u, b):
                pltpu.make_async_copy(wbufs[u], den_sh.at[dlr.at[b]],
                                      ssems[u]).wait()
                pltpu.make_async_copy(hbufs[u], out_sh.at[dlr.at[b]],
                                      ssems[u]).wait()

            def duo(p, c2):
                b0 = p * 2
                for u in range(2):
                    @pl.when(b0 + u < nblk)
                    def _():
                        issue_g(u, b0 + u)
                for u in range(2):
                    @pl.when(b0 + u < nblk)
                    def _():
                        wait_g(u, b0 + u)
                        mul(u)
                        issue_s(u, b0 + u)
                for u in range(2):
                    @pl.when(b0 + u < nblk)
                    def _():
                        wait_s(u, b0 + u)
                return c2
            lax.fori_loop(0, (nblk + 1) // 2, duo, 0)
            plsc.subcore_barrier()

            vrow = CH // 16
            pltpu.sync_copy(out_sh.at[pl.ds(sid * vrow, vrow)],
                            outp_hbm.at[cid, pl.ds(lo + sid * vrow, vrow)])
            pltpu.sync_copy(den_sh.at[pl.ds(sid * vrow, vrow)],
                            denp_hbm.at[cid, pl.ds(lo + sid * vrow, vrow)])
            plsc.subcore_barrier()
            return carry

        lax.fori_loop(0, NR, range_body, 0)

    return k(src_pad, dst_pad, w_full, h)


def _dense_back_body(o0_ref, o1_ref, d0_ref, d1_ref, wl_ref, h_ref,
                     bias_ref, fcw_ref, fcb_ref, o_ref):
    wl = wl_ref[...]
    den = d0_ref[0][:, :H] + d1_ref[0][:, :H] + wl
    num = o0_ref[0] + o1_ref[0] + h_ref[...] * jnp.repeat(wl, C, axis=1)
    val = num / jnp.repeat(den, C, axis=1) + bias_ref[...]
    val = jnp.maximum(val, 0.0)
    logits = jnp.dot(val, fcw_ref[...], preferred_element_type=jnp.float32) + fcb_ref[...]
    m = jnp.max(logits, axis=1, keepdims=True)
    z = logits - m
    lse = jnp.log(jnp.sum(jnp.exp(z), axis=1, keepdims=True))
    o_ref[...] = z - lse


def _dense_back(outp, denp, w_loop, h, bias, fc_W, fc_b):
    grid = (N // BLK,)
    return pl.pallas_call(
        _dense_back_body,
        grid=grid,
        in_specs=[
            pl.BlockSpec((1, BLK, HC), lambda i: (0, i, 0)),
            pl.BlockSpec((1, BLK, HC), lambda i: (1, i, 0)),
            pl.BlockSpec((1, BLK, 16), lambda i: (0, i, 0)),
            pl.BlockSpec((1, BLK, 16), lambda i: (1, i, 0)),
            pl.BlockSpec((BLK, H), lambda i: (i, 0)),
            pl.BlockSpec((BLK, HC), lambda i: (i, 0)),
            pl.BlockSpec((1, HC), lambda i: (0, 0)),
            pl.BlockSpec((HC, 5), lambda i: (0, 0)),
            pl.BlockSpec((1, 5), lambda i: (0, 0)),
        ],
        out_specs=pl.BlockSpec((BLK, 5), lambda i: (i, 0)),
        out_shape=jax.ShapeDtypeStruct((N, 5), jnp.float32),
    )(outp, outp, denp, denp, w_loop, h, bias, fc_W, fc_b)


def kernel(x, edge_index, W, att_src, att_dst, bias, fc_W, fc_b):
    src = edge_index[0]
    dst = edge_index[1]
    eyeH = jnp.eye(H, dtype=jnp.float32)
    As = (att_src.reshape(H, C, 1) * eyeH[:, None, :]).reshape(HC, H)
    Ad = (att_dst.reshape(H, C, 1) * eyeH[:, None, :]).reshape(HC, H)

    h, T1, T2, w_loop = _dense_front(x, W, As, Ad)

    padi = ((jnp.arange(NW * (EPW - ERW), dtype=jnp.int32) * 8191) % N
            ).reshape(NW, EPW - ERW)
    src_pad = jnp.concatenate([src.reshape(NW, ERW), padi], axis=1).reshape(EP)
    dst_pad = jnp.concatenate([dst.reshape(NW, ERW), padi], axis=1).reshape(EP)
    w_full = _sc_phase1(src_pad, dst_pad, T1, T2)
    outp, denp = _sc_phase2(src_pad, dst_pad, w_full, h)

    return _dense_back(outp, denp, w_loop, h,
                       bias.reshape(1, HC), fc_W, fc_b.reshape(1, 5))


# speedup vs baseline: 81.9415x; 1.0440x over previous
import functools
import jax
import jax.numpy as jnp
from jax import lax
from jax.experimental import pallas as pl
from jax.experimental.pallas import tpu as pltpu
from jax.experimental.pallas import tpu_sc as plsc

N = 100000
E = 1600000
H, C = 8, 16
IN = 16
HC = H * C
BLK = 2000

NW = 32
EB = 1024
NB1 = 49
EP = NW * NB1 * EB
EPW = EP // NW
ERW = E // NW

CH = 6144
NR = -(-N // CH)
NP = NR * CH
RPS = CH // 16
B2 = 128
ECAP = 3840
NQ = ECAP // B2
FCH = 112
NCHK = EPW // (FCH * 16)


def _dense_front_body(x_ref, w_ref, asm_ref, adm_ref, h_ref, t1_ref, t2_ref, wi_ref):
    x = x_ref[...]
    h = jnp.dot(x, w_ref[...], preferred_element_type=jnp.float32)
    a_src = jnp.dot(h, asm_ref[...], preferred_element_type=jnp.float32)
    a_dst = jnp.dot(h, adm_ref[...], preferred_element_type=jnp.float32)
    e = a_src + a_dst
    e = jnp.where(e >= 0, e, 0.2 * e)
    wi_ref[...] = jnp.exp(e)
    h_ref[...] = h
    t1_ref[...] = jnp.concatenate([a_src, a_dst], axis=1)
    t2_ref[...] = jnp.concatenate([a_dst, a_src], axis=1)


def _dense_front(x, W, As, Ad):
    grid = (N // BLK,)
    return pl.pallas_call(
        _dense_front_body,
        grid=grid,
        in_specs=[
            pl.BlockSpec((BLK, IN), lambda i: (i, 0)),
            pl.BlockSpec((IN, HC), lambda i: (0, 0)),
            pl.BlockSpec((HC, H), lambda i: (0, 0)),
            pl.BlockSpec((HC, H), lambda i: (0, 0)),
        ],
        out_specs=[
            pl.BlockSpec((BLK, HC), lambda i: (i, 0)),
            pl.BlockSpec((BLK, 2 * H), lambda i: (i, 0)),
            pl.BlockSpec((BLK, 2 * H), lambda i: (i, 0)),
            pl.BlockSpec((BLK, H), lambda i: (i, 0)),
        ],
        out_shape=[
            jax.ShapeDtypeStruct((N, HC), jnp.float32),
            jax.ShapeDtypeStruct((N, 2 * H), jnp.float32),
            jax.ShapeDtypeStruct((N, 2 * H), jnp.float32),
            jax.ShapeDtypeStruct((N, H), jnp.float32),
        ],
    )(x, W, As, Ad)


def _sc_phase1(src_pad, dst_pad, T1, T2):
    src2d = src_pad.reshape(EP // 128, 128)
    dst2d = dst_pad.reshape(EP // 128, 128)
    mesh = plsc.VectorSubcoreMesh(core_axis_name="c", subcore_axis_name="s",
                                  num_cores=2, num_subcores=16)

    @functools.partial(
        pl.kernel,
        out_type=jax.ShapeDtypeStruct((EP, 16), jnp.float32),
        mesh=mesh,
        scratch_types=[
            pltpu.VMEM((8, 128), jnp.int32),
            pltpu.VMEM((8, 128), jnp.int32),
            pltpu.VMEM((EB, 16), jnp.float32),
            pltpu.SemaphoreType.DMA,
        ],
        compiler_params=pltpu.CompilerParams(use_tc_tiling_on_sc=False),
    )
    def k(src_hbm, dst_hbm, t1_hbm, t2_hbm, w_hbm, sidx, didx, buf, sem):
        wid = lax.axis_index("s") * 2 + lax.axis_index("c")

        def blk(b, carry):
            g = wid * NB1 + b
            pltpu.sync_copy(src_hbm.at[pl.ds(g * 8, 8)], sidx)
            pltpu.sync_copy(dst_hbm.at[pl.ds(g * 8, 8)], didx)
            descs = [
                pltpu.async_copy(t1_hbm.at[sidx.at[j]],
                                 buf.at[pl.ds(j * 128, 128)], sem)
                for j in range(8)
            ]
            for d in descs:
                d.wait()
            descs = [
                pltpu.async_copy(t2_hbm.at[didx.at[j]],
                                 buf.at[pl.ds(j * 128, 128)], sem, add=True)
                for j in range(8)
            ]
            for d in descs:
                d.wait()

            def cbody(iv, c2):
                for u in range(16):
                    i = iv * 16 + u
                    v = buf[i, :]
                    v = jnp.where(v >= 0, v, 0.2 * v)
                    buf[i, :] = jnp.exp(v)
                return c2

            lax.fori_loop(0, EB // 16, cbody, 0)

            @pl.when(b == NB1 - 1)
            def _():
                def zpad(i, c2):
                    buf[(ERW - (NB1 - 1) * EB) + i, :] = jnp.zeros((16,), jnp.float32)
                    return c2
                lax.fori_loop(0, EB - (ERW - (NB1 - 1) * EB), zpad, 0)

            pltpu.sync_copy(buf, w_hbm.at[pl.ds(g * EB, EB)])
            return carry

        lax.fori_loop(0, NB1, blk, 0)

    return k(src2d, dst2d, T1, T2)


def _sc_phase2(src_pad, dst_pad, w_full, h):
    dst2d = dst_pad.reshape(EP // 16, 16)
    mesh = plsc.VectorSubcoreMesh(core_axis_name="c", subcore_axis_name="s",
                                  num_cores=2, num_subcores=16)

    @functools.partial(
        pl.kernel,
        out_type=[
            jax.ShapeDtypeStruct((2, NP, HC), jnp.float32),
            jax.ShapeDtypeStruct((2, NP, 16), jnp.float32),
        ],
        mesh=mesh,
        scratch_types=[
            pltpu.VMEM((ECAP,), jnp.int32),
            pltpu.VMEM((NQ, B2), jnp.int32),
            pltpu.VMEM((NQ, B2), jnp.int32),
            pltpu.VMEM((NQ, B2), jnp.int32),
            pltpu.VMEM((FCH * 16,), jnp.int32),
            pltpu.VMEM((FCH * 16,), jnp.int32),
            pltpu.VMEM((B2, 16), jnp.float32),
            pltpu.VMEM((B2, 16), jnp.float32),
            pltpu.VMEM((B2, HC), jnp.float32),
            pltpu.VMEM((B2, HC), jnp.float32),
            pltpu.VMEM((16, HC), jnp.float32),
            pltpu.VMEM((64, 16), jnp.float32),
            pltpu.VMEM_SHARED((CH, HC), jnp.float32),
            pltpu.VMEM_SHARED((CH, 16), jnp.float32),
            pltpu.SemaphoreType.DMA,
            pltpu.SemaphoreType.DMA,
            pltpu.SemaphoreType.DMA,
            pltpu.SemaphoreType.DMA,
            pltpu.SemaphoreType.DMA,
            pltpu.SemaphoreType.DMA,
            pltpu.SemaphoreType.DMA,
        ],
        compiler_params=pltpu.CompilerParams(use_tc_tiling_on_sc=False,
                                             needs_layout_passes=False),
    )
    def k(src_hbm, dst_flat_hbm, w_hbm, h_hbm, outp_hbm, denp_hbm,
          ebuf, srng, dsr, dlr, fb0, fb1, wb0, wb1, hb0, hb1,
          zbuf, zdbuf, out_sh, den_sh,
          psem, fs0, fs1, gs0, gs1, ss0, ss1):
        fbufs = (fb0, fb1)
        fsems = (fs0, fs1)
        wbufs = (wb0, wb1)
        hbufs = (hb0, hb1)
        gsems = (gs0, gs1)
        ssems = (ss0, ss1)
        cid = lax.axis_index("c")
        sid = lax.axis_index("s")
        wid = sid * 2 + cid
        ebase = wid * EPW
        iota = lax.iota(jnp.int32, 16)

        def zv(buf, nrow, ncol):
            def zi(i, c):
                for kk in range(ncol // 16):
                    buf[i, pl.ds(kk * 16, 16)] = jnp.zeros((16,), jnp.float32)
                return c
            lax.fori_loop(0, nrow, zi, 0)

        zv(zbuf, 16, HC)
        zv(zdbuf, 64, 16)
        fchunk = lambda c: dst_flat_hbm.at[pl.ds(wid * EPW + c * FCH * 16,
                                                 FCH * 16)]

        def range_body(r, carry):
            lo = r * CH
            row0 = sid * RPS
            for kk in range(RPS // 16):
                pltpu.sync_copy(zbuf, out_sh.at[pl.ds(row0 + kk * 16, 16)])
            for kk in range(RPS // 64):
                pltpu.sync_copy(zdbuf, den_sh.at[pl.ds(row0 + kk * 64, 64)])
            plsc.subcore_barrier()

            lov = jnp.full((16,), lo, jnp.int32)
            ebasev = jnp.full((16,), ebase, jnp.int32)
            erealv = ebasev + ERW

            for u in range(2):
                pltpu.async_copy(fchunk(u), fbufs[u], fsems[u])

            def fchunk_body(c, u, cnt):
                pltpu.make_async_copy(fchunk(c), fbufs[u], fsems[u]).wait()

                def fbody(v, cnt2):
                    d = fbufs[u][pl.ds(v * 16, 16)]
                    eidv = iota + jnp.full(
                        (16,), c * FCH * 16 + v * 16 + ebase, jnp.int32)
                    m = (d >= lov) & (d < lov + CH) & (eidv < erealv)
                    cs = plsc.cumsum(m.astype(jnp.int32))
                    pos = jnp.full((16,), cnt2, jnp.int32) + cs - 1
                    plsc.store_scatter(ebuf, [pos], eidv, mask=m)
                    return cnt2 + cs[15]
                cnt = lax.fori_loop(0, FCH, fbody, cnt)

                @pl.when(c + 2 < NCHK)
                def _():
                    pltpu.async_copy(fchunk(c + 2), fbufs[u], fsems[u])
                return cnt

            def fpair(p, cnt):
                for u in range(2):
                    cnt = fchunk_body(p * 2 + u, u, cnt)
                return cnt
            cnt = lax.fori_loop(0, NCHK // 2, fpair, jnp.int32(0))
            padv = erealv + (iota & 7) * 8
            for kk in range(B2 // 16):
                ebuf[pl.ds(cnt + kk * 16, 16)] = padv
            nblk = (cnt + (B2 - 1)) // B2
            cntv = jnp.full((16,), cnt, jnp.int32)

            for q in range(NQ):
                @pl.when(q < nblk)
                def _():
                    pltpu.async_copy(src_hbm.at[ebuf.at[pl.ds(q * B2, B2)]],
                                     srng.at[q], psem)
                    pltpu.async_copy(dst_flat_hbm.at[ebuf.at[pl.ds(q * B2, B2)]],
                                     dsr.at[q], psem)
            for q in range(NQ):
                @pl.when(q < nblk)
                def _():
                    pltpu.make_async_copy(src_hbm.at[ebuf.at[pl.ds(q * B2, B2)]],
                                          srng.at[q], psem).wait()
                    pltpu.make_async_copy(dst_flat_hbm.at[ebuf.at[pl.ds(q * B2, B2)]],
                                          dsr.at[q], psem).wait()

            def dbody(bq, c2):
                for j in range(B2 // 16):
                    dg = dsr[bq, pl.ds(j * 16, 16)]
                    gpos = iota + jnp.full((16,), bq * B2 + j * 16, jnp.int32)
                    dl = jnp.where(gpos < cntv, dg - lov, iota & 7)
                    dlr[bq, pl.ds(j * 16, 16)] = dl
                return c2
            lax.fori_loop(0, nblk, dbody, 0)

            def issue_g(u, b):
                pltpu.async_copy(w_hbm.at[ebuf.at[pl.ds(b * B2, B2)]],
                                 wbufs[u], gsems[u])
                pltpu.async_copy(h_hbm.at[srng.at[b]], hbufs[u], gsems[u])

            def wait_g(u, b):
                pltpu.make_async_copy(w_hbm.at[ebuf.at[pl.ds(b * B2, B2)]],
                                      wbufs[u], gsems[u]).wait()
                pltpu.make_async_copy(h_hbm.at[srng.at[b]], hbufs[u],
                                      gsems[u]).wait()

            def mul(u):
                wbu, hbu = wbufs[u], hbufs[u]

                def mbody(i2, c3):
                    for e in range(2):
                        i = i2 * 2 + e
                        wrow = wbu[i, :]
                        for hh in range(H):
                            hidx = jnp.full((16,), hh, jnp.int32)
                            wb = wrow.at[hidx].get(mode="promise_in_bounds")
                            hv = hbu[i, pl.ds(hh * C, C)]
                            hbu[i, pl.ds(hh * C, C)] = hv * wb
                    return c3
                lax.fori_loop(0, B2 // 2, mbody, 0)

            def issue_s(u, b):
                pltpu.async_copy(wbufs[u], den_sh.at[dlr.at[b]], ssems[u],
                                 add=True)
                pltpu.async_copy(hbufs[u], out_sh.at[dlr.at[b]], ssems[u],
                                 add=True)

            def wait_s(u, b):
                pltpu.make_async_copy(wbufs[u], den_sh.at[dlr.at[b]],
                                      ssems[u]).wait()
                pltpu.make_async_copy(hbufs[u], out_sh.at[dlr.at[b]],
                                      ssems[u]).wait()

            def duo(p, c2):
                b0 = p * 2
                for u in range(2):
                    @pl.when(b0 + u < nblk)
                    def _():
                        issue_g(u, b0 + u)
                for u in range(2):
                    @pl.when(b0 + u < nblk)
                    def _():
                        wait_g(u, b0 + u)
                        mul(u)
                        issue_s(u, b0 + u)
                for u in range(2):
                    @pl.when(b0 + u < nblk)
                    def _():
                        wait_s(u, b0 + u)
                return c2
            lax.fori_loop(0, (nblk + 1) // 2, duo, 0)
            plsc.subcore_barrier()

            vrow = CH // 16
            pltpu.sync_copy(out_sh.at[pl.ds(sid * vrow, vrow)],
                            outp_hbm.at[cid, pl.ds(lo + sid * vrow, vrow)])
            pltpu.sync_copy(den_sh.at[pl.ds(sid * vrow, vrow)],
                            denp_hbm.at[cid, pl.ds(lo + sid * vrow, vrow)])
            plsc.subcore_barrier()
            return carry

        lax.fori_loop(0, NR, range_body, 0)

    return k(src_pad, dst_pad, w_full, h)


def _dense_back_body(o0_ref, o1_ref, d0_ref, d1_ref, wl_ref, h_ref,
                     bias_ref, fcw_ref, fcb_ref, o_ref):
    wl = wl_ref[...]
    den = d0_ref[0][:, :H] + d1_ref[0][:, :H] + wl
    num = o0_ref[0] + o1_ref[0] + h_ref[...] * jnp.repeat(wl, C, axis=1)
    val = num / jnp.repeat(den, C, axis=1) + bias_ref[...]
    val = jnp.maximum(val, 0.0)
    logits = jnp.dot(val, fcw_ref[...], preferred_element_type=jnp.float32) + fcb_ref[...]
    m = jnp.max(logits, axis=1, keepdims=True)
    z = logits - m
    lse = jnp.log(jnp.sum(jnp.exp(z), axis=1, keepdims=True))
    o_ref[...] = z - lse


def _dense_back(outp, denp, w_loop, h, bias, fc_W, fc_b):
    grid = (N // BLK,)
    return pl.pallas_call(
        _dense_back_body,
        grid=grid,
        in_specs=[
            pl.BlockSpec((1, BLK, HC), lambda i: (0, i, 0)),
            pl.BlockSpec((1, BLK, HC), lambda i: (1, i, 0)),
            pl.BlockSpec((1, BLK, 16), lambda i: (0, i, 0)),
            pl.BlockSpec((1, BLK, 16), lambda i: (1, i, 0)),
            pl.BlockSpec((BLK, H), lambda i: (i, 0)),
            pl.BlockSpec((BLK, HC), lambda i: (i, 0)),
            pl.BlockSpec((1, HC), lambda i: (0, 0)),
            pl.BlockSpec((HC, 5), lambda i: (0, 0)),
            pl.BlockSpec((1, 5), lambda i: (0, 0)),
        ],
        out_specs=pl.BlockSpec((BLK, 5), lambda i: (i, 0)),
        out_shape=jax.ShapeDtypeStruct((N, 5), jnp.float32),
    )(outp, outp, denp, denp, w_loop, h, bias, fc_W, fc_b)


def kernel(x, edge_index, W, att_src, att_dst, bias, fc_W, fc_b):
    src = edge_index[0]
    dst = edge_index[1]
    eyeH = jnp.eye(H, dtype=jnp.float32)
    As = (att_src.reshape(H, C, 1) * eyeH[:, None, :]).reshape(HC, H)
    Ad = (att_dst.reshape(H, C, 1) * eyeH[:, None, :]).reshape(HC, H)

    h, T1, T2, w_loop = _dense_front(x, W, As, Ad)

    padi = ((jnp.arange(NW * (EPW - ERW), dtype=jnp.int32) * 8191) % N
            ).reshape(NW, EPW - ERW)
    src_pad = jnp.concatenate([src.reshape(NW, ERW), padi], axis=1).reshape(EP)
    dst_pad = jnp.concatenate([dst.reshape(NW, ERW), padi], axis=1).reshape(EP)
    w_full = _sc_phase1(src_pad, dst_pad, T1, T2)
    outp, denp = _sc_phase2(src_pad, dst_pad, w_full, h)

    return _dense_back(outp, denp, w_loop, h,
                       bias.reshape(1, HC), fc_W, fc_b.reshape(1, 5))
